# Initial kernel scaffold; baseline (speedup 1.0000x reference)
import jax
import jax.numpy as jnp
from jax.experimental import pallas as pl


def kernel(x_person, x_household, x_zone, x_purpose, ei_belongs_to, ei_located_in, ei_has_purpose, ei_performs, ei_parent, ei_child, ei_spouse, ei_housemate, ei_sibling, Ws, Wd, att_s, att_d, bias):
    raise NotImplementedError("write your pallas kernel here")


# speedup vs baseline: 1.3660x; 1.3660x over previous
import functools

import jax
import jax.numpy as jnp
from jax import lax
from jax.experimental import pallas as pl
from jax.experimental.pallas import tpu as pltpu

D = 128
H = 64
NP_, NH, NZ, NU = 100000, 50000, 10000, 4096

REL_NS = [NP_, NH, NZ, NP_, NP_, NP_, NP_, NP_, NP_]
REL_ND = [NH, NZ, NU, NU, NP_, NP_, NP_, NP_, NP_]

PERSON_SRC_RELS = (0, 3, 4, 5, 6, 7, 8)


def _matmul_body(x_ref, w_ref, o_ref):
    o_ref[...] = jnp.dot(x_ref[...], w_ref[...],
                         preferred_element_type=jnp.float32)


def _matmul(x, w, bm=1024):
    m, k = x.shape
    n = w.shape[1]
    grid = (pl.cdiv(m, bm),)
    return pl.pallas_call(
        _matmul_body,
        grid=grid,
        in_specs=[
            pl.BlockSpec((bm, k), lambda i: (i, 0)),
            pl.BlockSpec((k, n), lambda i: (0, 0)),
        ],
        out_specs=pl.BlockSpec((bm, n), lambda i: (i, 0)),
        out_shape=jax.ShapeDtypeStruct((m, n), jnp.float32),
    )(x, w)


def kernel(x_person, x_household, x_zone, x_purpose, ei_belongs_to,
           ei_located_in, ei_has_purpose, ei_performs, ei_parent, ei_child,
           ei_spouse, ei_housemate, ei_sibling, Ws, Wd, att_s, att_d, bias):
    eis = [ei_belongs_to, ei_located_in, ei_has_purpose, ei_performs,
           ei_parent, ei_child, ei_spouse, ei_housemate, ei_sibling]

    vs = jnp.einsum('rdh,rh->rd', Ws, att_s)
    vd = jnp.einsum('rdh,rh->rd', Wd, att_d)

    w_p = jnp.concatenate(
        [Ws[i] for i in PERSON_SRC_RELS]
        + [vs[jnp.array(PERSON_SRC_RELS)].T, vd[4:9].T], axis=1)
    p_out = _matmul(x_person, w_p)
    hs_p = {r: p_out[:, 64 * j:64 * (j + 1)]
            for j, r in enumerate(PERSON_SRC_RELS)}
    es_p = {r: p_out[:, 448 + j] for j, r in enumerate(PERSON_SRC_RELS)}
    ed_p = {r: p_out[:, 455 + (r - 4)] for r in (4, 5, 6, 7, 8)}

    w_h = jnp.concatenate([Ws[1], vs[1][:, None], vd[0][:, None]], axis=1)
    h_out = _matmul(x_household, w_h)
    w_z = jnp.concatenate([Ws[2], vs[2][:, None], vd[1][:, None]], axis=1)
    z_out = _matmul(x_zone, w_z)
    u_out = _matmul(x_purpose, vd[2:4].T)

    hs = {**hs_p, 1: h_out[:, :64], 2: z_out[:, :64]}
    es = {**es_p, 1: h_out[:, 64], 2: z_out[:, 64]}
    ed = {**ed_p, 0: h_out[:, 65], 1: z_out[:, 65], 2: u_out[:, 0],
          3: u_out[:, 1]}

    def rel_out(r):
        src, dst = eis[r][0], eis[r][1]
        e = es[r][src] + ed[r][dst]
        e = jnp.where(e > 0, e, 0.2 * e)
        ex = jnp.exp(e)
        den = jax.ops.segment_sum(ex, dst, num_segments=REL_ND[r])
        alpha = ex / (den[dst] + 1e-16)
        num = jax.ops.segment_sum(alpha[:, None] * hs[r][src], dst,
                                  num_segments=REL_ND[r])
        return num + bias[r]

    hh = rel_out(0)
    zz = rel_out(1)
    pu = rel_out(2) + rel_out(3)
    pe = rel_out(4) + rel_out(5) + rel_out(6) + rel_out(7) + rel_out(8)
    return (jax.nn.relu(pe), jax.nn.relu(hh), jax.nn.relu(zz),
            jax.nn.relu(pu))


# speedup vs baseline: 2.3943x; 1.7528x over previous
import functools

import jax
import jax.numpy as jnp
from jax import lax
from jax.experimental import pallas as pl
from jax.experimental.pallas import tpu as pltpu
from jax.experimental.pallas import tpu_sc as plsc

D = 128
H = 64
NP_, NH, NZ, NU = 100000, 50000, 10000, 4096

REL_NS = [NP_, NH, NZ, NP_, NP_, NP_, NP_, NP_, NP_]
REL_ND = [NH, NZ, NU, NU, NP_, NP_, NP_, NP_, NP_]
REL_NE = [100000, 50000, 50000, 150000, 60000, 60000, 40000, 60000, 50000]
PERSON_SRC_RELS = (0, 3, 4, 5, 6, 7, 8)

NC, NS = 2, 16
NW = NC * NS
B = 256

REL_EP = [((e + B - 1) // B) * B for e in REL_NE]
REL_EOFF = [sum(REL_EP[:i]) for i in range(9)]
ET = sum(REL_EP)

DEN_OFF = [0]
for _nd in REL_ND[:-1]:
    DEN_OFF.append(DEN_OFF[-1] + _nd)
NDEN = DEN_OFF[-1] + REL_ND[-1]
NDENP = ((NDEN + NS * 128 - 1) // (NS * 128)) * (NS * 128)
ZLEN = NDENP // NS


def _pad128(n):
    return ((n + 127) // 128) * 128


NPP, NHP, NZP, NUP = _pad128(NP_), _pad128(NH), _pad128(NZ), _pad128(NU)

ES_COL = {0: 0, 3: 1, 4: 2, 5: 3, 6: 4, 7: 5, 8: 6, 1: 0, 2: 0}
ED_COL = {4: 7, 5: 8, 6: 9, 7: 10, 8: 11, 0: 1, 1: 1, 2: 0, 3: 1}

REL_TAB = {0: (0, 0), 3: (0, 64), 4: (1, 0), 5: (1, 64), 6: (2, 0),
           7: (2, 64), 8: (3, 0), 1: (4, 0), 2: (5, 0)}

_f32 = jnp.float32
_i32 = jnp.int32


def _person_mm_body(x_ref, w_ref, scal_ref, t03_ref, t45_ref, t67_ref,
                    t8_ref):
    res = jnp.dot(x_ref[...], w_ref[...], preferred_element_type=_f32)
    bm = res.shape[0]
    scal_ref[...] = jnp.concatenate(
        [res[:, 448:460], jnp.zeros((bm, 4), _f32)], axis=1)
    t03_ref[...] = res[:, 0:128]
    t45_ref[...] = res[:, 128:256]
    t67_ref[...] = res[:, 256:384]
    t8_ref[...] = jnp.concatenate(
        [res[:, 384:448], jnp.zeros((bm, 64), _f32)], axis=1)


def _person_matmuls(x_person, w_p, bm=1024):
    grid = (pl.cdiv(NP_, bm),)
    outs = ([jax.ShapeDtypeStruct((NP_, 16), _f32)]
            + [jax.ShapeDtypeStruct((NP_, 128), _f32)] * 4)
    specs = ([pl.BlockSpec((bm, 16), lambda i: (i, 0))]
             + [pl.BlockSpec((bm, 128), lambda i: (i, 0))] * 4)
    return pl.pallas_call(
        _person_mm_body,
        grid=grid,
        in_specs=[pl.BlockSpec((bm, 128), lambda i: (i, 0)),
                  pl.BlockSpec((128, 460), lambda i: (0, 0))],
        out_specs=specs,
        out_shape=outs,
    )(x_person, w_p)


def _small_mm_body(ncols, x_ref, w_ref, hs_ref, scal_ref):
    res = jnp.dot(x_ref[...], w_ref[...], preferred_element_type=_f32)
    bm = res.shape[0]
    hs_ref[...] = jnp.concatenate(
        [res[:, 0:64], jnp.zeros((bm, 64), _f32)], axis=1)
    scal_ref[...] = jnp.concatenate(
        [res[:, 64:64 + ncols], jnp.zeros((bm, 16 - ncols), _f32)], axis=1)


def _small_matmul(x, w, ncols, bm=1024):
    m = x.shape[0]
    n = w.shape[1]
    grid = (pl.cdiv(m, bm),)
    return pl.pallas_call(
        functools.partial(_small_mm_body, ncols),
        grid=grid,
        in_specs=[pl.BlockSpec((bm, 128), lambda i: (i, 0)),
                  pl.BlockSpec((128, n), lambda i: (0, 0))],
        out_specs=[pl.BlockSpec((bm, 128), lambda i: (i, 0)),
                   pl.BlockSpec((bm, 16), lambda i: (i, 0))],
        out_shape=[jax.ShapeDtypeStruct((m, 128), _f32),
                   jax.ShapeDtypeStruct((m, 16), _f32)],
    )(x, w)


def _purpose_mm_body(x_ref, w_ref, scal_ref):
    res = jnp.dot(x_ref[...], w_ref[...], preferred_element_type=_f32)
    bm = res.shape[0]
    scal_ref[...] = jnp.concatenate(
        [res[:, 0:2], jnp.zeros((bm, 14), _f32)], axis=1)


def _purpose_matmul(x, w, bm=1024):
    m = x.shape[0]
    return pl.pallas_call(
        _purpose_mm_body,
        grid=(pl.cdiv(m, bm),),
        in_specs=[pl.BlockSpec((bm, 128), lambda i: (i, 0)),
                  pl.BlockSpec((128, 2), lambda i: (0, 0))],
        out_specs=pl.BlockSpec((bm, 16), lambda i: (i, 0)),
        out_shape=jax.ShapeDtypeStruct((m, 16), _f32),
    )(x, w)


_MESH = plsc.VectorSubcoreMesh(core_axis_name="c", subcore_axis_name="s")


def _worker_id():
    c = lax.axis_index("c")
    s = lax.axis_index("s")
    return c, s, s * NC + c


def _chunk_loop(ep, wid, body):
    nch = ep // B
    trip = jnp.maximum((nch - wid + NW - 1) // NW, 0)

    def outer(k, _):
        body((wid + k * NW) * B)
        return 0

    lax.fori_loop(0, trip, outer, 0)


def _zero_vmem_1d(ref, n):
    def zb(j, _):
        ref[pl.ds(j * 16, 16)] = jnp.zeros((16,), _f32)
        return 0
    lax.fori_loop(0, n // 16, zb, 0)


def _copy_1d(src_at, dst_at, total, step):
    k = 0
    while k * step < total:
        ln = min(step, total - k * step)
        pltpu.sync_copy(src_at(k * step, ln), dst_at(k * step, ln))
        k += 1


def _copy_rows_loop(src_at, dst_at, rpt, zr):
    full = rpt // zr
    rem = rpt - full * zr
    if full:
        def cp(k, _):
            pltpu.sync_copy(src_at(k * zr, zr), dst_at(k * zr, zr))
            return 0
        lax.fori_loop(0, full, cp, 0)
    if rem:
        pltpu.sync_copy(src_at(full * zr, rem), dst_at(full * zr, rem))


def _kernel_s_body(*refs):
    (srcs, dsts) = refs[0:9], refs[9:18]
    ess, eds = refs[18:27], refs[27:36]
    ex_out, den0_out, den1_out = refs[36:39]
    (srcbuf, dstbuf, dstobuf, exbuf, esbuf, edbuf, zbuf, den_sh) = refs[39:47]

    c, s, wid = _worker_id()
    iota16 = lax.iota(_i32, 16)

    _zero_vmem_1d(zbuf, 8192)
    _copy_1d(lambda o, ln: zbuf.at[pl.ds(0, ln)],
             lambda o, ln: den_sh.at[pl.ds(s * ZLEN + o, ln)],
             ZLEN, 8192)
    plsc.subcore_barrier()

    for r in range(9):
        ne, off = REL_NE[r], DEN_OFF[r]

        def chunk(lo, r=r, ne=ne, off=off):
            pltpu.sync_copy(srcs[r].at[pl.ds(lo, B)], srcbuf)
            pltpu.sync_copy(dsts[r].at[pl.ds(lo, B)], dstbuf)
            pltpu.sync_copy(ess[r].at[srcbuf], esbuf)
            pltpu.sync_copy(eds[r].at[dstbuf], edbuf)

            def inner(j, _):
                sl = pl.ds(j * 16, 16)
                e = esbuf[sl] + edbuf[sl]
                e = jnp.where(e > 0, e, 0.2 * e)
                ex = jnp.exp(e)
                ex = jnp.where(lo + iota16 + j * 16 < ne, ex, 0.0)
                exbuf[sl] = ex
                dstobuf[sl] = dstbuf[sl] + off
                return 0

            lax.fori_loop(0, B // 16, inner, 0)
            pltpu.sync_copy(exbuf, ex_out.at[pl.ds(REL_EOFF[r] + lo, B)])
            pltpu.sync_copy(exbuf, den_sh.at[dstobuf], add=True)

        _chunk_loop(REL_EP[r], wid, chunk)

    plsc.subcore_barrier()

    @pl.when(c == 0)
    def _():
        _copy_1d(lambda o, ln: den_sh.at[pl.ds(s * ZLEN + o, ln)],
                 lambda o, ln: den0_out.at[pl.ds(s * ZLEN + o, ln)],
                 ZLEN, 8192)

    @pl.when(c == 1)
    def _():
        _copy_1d(lambda o, ln: den_sh.at[pl.ds(s * ZLEN + o, ln)],
                 lambda o, ln: den1_out.at[pl.ds(s * ZLEN + o, ln)],
                 ZLEN, 8192)


def _run_kernel_s(srcs, dsts, ess, eds):
    f = pl.kernel(
        _kernel_s_body,
        out_type=[jax.ShapeDtypeStruct((ET,), _f32),
                  jax.ShapeDtypeStruct((NDENP,), _f32),
                  jax.ShapeDtypeStruct((NDENP,), _f32)],
        mesh=_MESH,
        scratch_types=[
            pltpu.VMEM((B,), _i32), pltpu.VMEM((B,), _i32),
            pltpu.VMEM((B,), _i32), pltpu.VMEM((B,), _f32),
            pltpu.VMEM((B,), _f32), pltpu.VMEM((B,), _f32),
            pltpu.VMEM((8192,), _f32),
            pltpu.VMEM_SHARED((NDENP,), _f32),
        ],
    )
    return f(*srcs, *dsts, *ess, *eds)


def _kernel_da_body(*refs):
    srcs, dsts = refs[0:9], refs[9:18]
    ex_in, den0, den1 = refs[18:21]
    tabs = refs[21:27]
    gouts = refs[27:31]
    (srcbuf, dstbuf, dstobuf, exb, d0b, d1b, ab, rows,
     g0, g1, g2, g3) = refs[31:43]
    gbufs = (g0, g1, g2, g3)

    c, s, wid = _worker_id()

    for r in range(9):
        off = DEN_OFF[r]
        tab_i, half = REL_TAB[r]
        tab = tabs[tab_i]

        def chunk(lo, r=r, off=off, tab=tab, half=half):
            pltpu.sync_copy(srcs[r].at[pl.ds(lo, B)], srcbuf)
            pltpu.sync_copy(dsts[r].at[pl.ds(lo, B)], dstbuf)
            pltpu.sync_copy(ex_in.at[pl.ds(REL_EOFF[r] + lo, B)], exb)

            def addoff(j, _):
                dstobuf[pl.ds(j * 16, 16)] = dstbuf[pl.ds(j * 16, 16)] + off
                return 0

            lax.fori_loop(0, B // 16, addoff, 0)
            pltpu.sync_copy(den0.at[dstobuf], d0b)
            pltpu.sync_copy(den1.at[dstobuf], d1b)
            pltpu.sync_copy(tab.at[srcbuf], rows)

            def scale(j, _):
                sl = pl.ds(j * 16, 16)
                av = exb[sl] / (d0b[sl] + d1b[sl] + 1e-16)
                ab[sl] = av
                for i in range(16):
                    erow = j * 16 + i
                    for c4 in range(4):
                        v = rows[erow, pl.ds(half + 16 * c4, 16)] * av[i]
                        gbufs[c4][pl.ds(256 * j + 16 * i, 16)] = v
                return 0

            lax.fori_loop(0, B // 16, scale, 0)
            gb = (REL_EOFF[r] + lo) * 16
            for c4 in range(4):
                pltpu.sync_copy(gbufs[c4],
                                gouts[c4].at[pl.ds(gb, B * 16)])

        _chunk_loop(REL_EP[r], wid, chunk)


def _run_kernel_da(srcs, dsts, ex_all, den0, den1, tabs):
    f = pl.kernel(
        _kernel_da_body,
        out_type=[jax.ShapeDtypeStruct((ET * 16,), _f32)] * 4,
        mesh=_MESH,
        scratch_types=[
            pltpu.VMEM((B,), _i32), pltpu.VMEM((B,), _i32),
            pltpu.VMEM((B,), _i32), pltpu.VMEM((B,), _f32),
            pltpu.VMEM((B,), _f32), pltpu.VMEM((B,), _f32),
            pltpu.VMEM((B,), _f32), pltpu.VMEM((B, 128), _f32),
            pltpu.VMEM((B * 16,), _f32), pltpu.VMEM((B * 16,), _f32),
            pltpu.VMEM((B * 16,), _f32), pltpu.VMEM((B * 16,), _f32),
        ],
    )
    return f(*srcs, *dsts, ex_all, den0, den1, *tabs)


HROWS = NPP // 2
JROW = HROWS
ACC_ROWS = HROWS + 8


def _kernel_a2_body(phases, nrel, *refs):
    dsts = refs[0:nrel]
    gins = refs[nrel:nrel + 4]
    nouts = len({p[0] for p in phases})
    outs = refs[nrel + 4:nrel + 4 + nouts]
    dstbuf, idxbuf, gflat, zbuf, acc = refs[nrel + 4 + nouts:]

    c, s, wid = _worker_id()
    iota16 = lax.iota(_i32, 16)

    _zero_vmem_1d(zbuf, 8192)

    for oi, base, rows_n, out_off, rel_list in phases:
        rpt16 = (rows_n // NS) * 16
        ndp_out = outs[oi].shape[0] // 32
        _copy_1d(lambda o, ln: zbuf.at[pl.ds(0, ln)],
                 lambda o, ln, s=s, rpt16=rpt16:
                     acc.at[pl.ds(s * rpt16 + o, ln)],
                 rpt16, 8192)
        plsc.subcore_barrier()

        for ri, r, c4 in rel_list:
            def chunk(lo, ri=ri, r=r, c4=c4, base=base, rows_n=rows_n):
                pltpu.sync_copy(dsts[ri].at[pl.ds(lo, B)], dstbuf)
                gb = (REL_EOFF[r] + lo) * 16
                pltpu.sync_copy(gins[c4].at[pl.ds(gb, B * 16)], gflat)

                def mkidx(j, _):
                    sl = pl.ds(j * 16, 16)
                    v = dstbuf[sl]
                    if base is not None:
                        vb = v - base
                        ok = (vb >= 0) & (vb < rows_n)
                        v = jnp.where(ok, vb, JROW)
                    v16 = v * 16
                    for i in range(16):
                        idxbuf[pl.ds(256 * j + 16 * i, 16)] = v16[i] + iota16
                    return 0

                lax.fori_loop(0, B // 16, mkidx, 0)
                pltpu.sync_copy(gflat, acc.at[idxbuf], add=True)

            _chunk_loop(REL_EP[r], wid, chunk)

        plsc.subcore_barrier()
        _copy_1d(lambda o, ln, s=s, rpt16=rpt16:
                     acc.at[pl.ds(s * rpt16 + o, ln)],
                 lambda o, ln, oi=oi, ndp=ndp_out, oo=out_off, s=s,
                        rpt16=rpt16:
                     outs[oi].at[pl.ds((c * ndp + oo) * 16 + s * rpt16 + o,
                                       ln)],
                 rpt16, 8192)
        plsc.subcore_barrier()


def _run_kernel_a2_group(dsts_sub, gins, phases, ndp_out, nouts):
    f = pl.kernel(
        functools.partial(_kernel_a2_body, phases, len(dsts_sub)),
        out_type=[jax.ShapeDtypeStruct((2 * ndp_out * 16,), _f32)] * nouts,
        mesh=_MESH,
        scratch_types=[
            pltpu.VMEM((B,), _i32),
            pltpu.VMEM((B * 16,), _i32),
            pltpu.VMEM((B * 16,), _f32),
            pltpu.VMEM((8192,), _f32),
            pltpu.VMEM_SHARED((ACC_ROWS * 16,), _f32),
        ],
    )
    return f(*dsts_sub, *gins)


def _run_kernel_a2(dsts, gins):
    pe_phases = [(c, hf * HROWS, HROWS, hf * HROWS,
                  [(i, r, c) for i, r in enumerate(range(4, 9))])
                 for c in range(4) for hf in range(2)]
    pe = _run_kernel_a2_group([dsts[r] for r in range(4, 9)], gins,
                              pe_phases, NPP, 4)
    hh = _run_kernel_a2_group([dsts[0]], gins,
                              [(c, None, NHP, 0, [(0, 0, c)])
                               for c in range(4)], NHP, 4)
    zz = _run_kernel_a2_group([dsts[1]], gins,
                              [(c, None, NZP, 0, [(0, 1, c)])
                               for c in range(4)], NZP, 4)
    pu = _run_kernel_a2_group([dsts[2], dsts[3]], gins,
                              [(c, None, NUP, 0, [(0, 2, c), (1, 3, c)])
                               for c in range(4)], NUP, 4)
    return list(pe) + list(hh) + list(zz) + list(pu)


def _epi_body(bias_ref, *refs):
    chunks, o_ref = refs[:-1], refs[-1]
    for ci in range(4):
        n = chunks[ci]
        part = n[0] + n[1]
        o_ref[:, pl.ds(ci * 16, 16)] = jnp.maximum(
            part + bias_ref[0, pl.ds(ci * 16, 16)], 0.0)


def _epilogue(parts, nd, ndp, bias_row, bm=2048):
    parts = [p.reshape(2, ndp, 16) for p in parts]
    return pl.pallas_call(
        _epi_body,
        grid=(pl.cdiv(nd, bm),),
        in_specs=[pl.BlockSpec((1, 64), lambda i: (0, 0))]
        + [pl.BlockSpec((2, bm, 16), lambda i: (0, i, 0))] * 4,
        out_specs=pl.BlockSpec((bm, 64), lambda i: (i, 0)),
        out_shape=jax.ShapeDtypeStruct((nd, 64), _f32),
    )(bias_row, *parts)


def kernel(x_person, x_household, x_zone, x_purpose, ei_belongs_to,
           ei_located_in, ei_has_purpose, ei_performs, ei_parent, ei_child,
           ei_spouse, ei_housemate, ei_sibling, Ws, Wd, att_s, att_d, bias):
    eis = [ei_belongs_to, ei_located_in, ei_has_purpose, ei_performs,
           ei_parent, ei_child, ei_spouse, ei_housemate, ei_sibling]

    vs = jnp.einsum('rdh,rh->rd', Ws, att_s)
    vd = jnp.einsum('rdh,rh->rd', Wd, att_d)

    w_p = jnp.concatenate(
        [Ws[i] for i in PERSON_SRC_RELS]
        + [vs[jnp.array(PERSON_SRC_RELS)].T, vd[4:9].T], axis=1)
    w_h = jnp.concatenate([Ws[1], vs[1][:, None], vd[0][:, None]], axis=1)
    w_z = jnp.concatenate([Ws[2], vs[2][:, None], vd[1][:, None]], axis=1)
    w_u = vd[2:4].T

    t_p, t03, t45, t67, t8 = _person_matmuls(x_person, w_p)
    th, t_h = _small_matmul(x_household, w_h, 2)
    tz, t_z = _small_matmul(x_zone, w_z, 2)
    t_u = _purpose_matmul(x_purpose, w_u)
    tabs = [t03, t45, t67, t8, th, tz]

    srcs, dsts = [], []
    for r in range(9):
        pad = REL_EP[r] - REL_NE[r]
        srcs.append(jnp.concatenate([eis[r][0], jnp.zeros((pad,), _i32)]))
        dsts.append(jnp.concatenate([eis[r][1], jnp.zeros((pad,), _i32)]))

    scal_src = [t_p, t_h, t_z, t_p, t_p, t_p, t_p, t_p, t_p]
    scal_dst = [t_h, t_z, t_u, t_u, t_p, t_p, t_p, t_p, t_p]
    ess = [scal_src[r][:, ES_COL[r]] for r in range(9)]
    eds = [scal_dst[r][:, ED_COL[r]] for r in range(9)]

    ex_all, den0, den1 = _run_kernel_s(srcs, dsts, ess, eds)
    gchunks = _run_kernel_da(srcs, dsts, ex_all, den0, den1, tabs)
    nums = _run_kernel_a2(dsts, gchunks)

    bias_pe = jnp.sum(bias[4:9], axis=0)[None, :]
    pe = _epilogue(list(nums[0:4]), NP_, NPP, bias_pe)
    hh = _epilogue(list(nums[4:8]), NH, NHP, bias[0][None, :])
    zz = _epilogue(list(nums[8:12]), NZ, NZP, bias[1][None, :])
    pu = _epilogue(list(nums[12:16]), NU, NUP, (bias[2] + bias[3])[None, :])
    return (pe, hh, zz, pu)


# speedup vs baseline: 2.5265x; 1.0552x over previous
import functools

import jax
import jax.numpy as jnp
from jax import lax
from jax.experimental import pallas as pl
from jax.experimental.pallas import tpu as pltpu
from jax.experimental.pallas import tpu_sc as plsc

D = 128
H = 64
NP_, NH, NZ, NU = 100000, 50000, 10000, 4096

REL_NS = [NP_, NH, NZ, NP_, NP_, NP_, NP_, NP_, NP_]
REL_ND = [NH, NZ, NU, NU, NP_, NP_, NP_, NP_, NP_]
REL_NE = [100000, 50000, 50000, 150000, 60000, 60000, 40000, 60000, 50000]
PERSON_SRC_RELS = (0, 3, 4, 5, 6, 7, 8)

NC, NS = 2, 16
NW = NC * NS
B = 512

REL_EP = [((e + B - 1) // B) * B for e in REL_NE]
REL_EOFF = [sum(REL_EP[:i]) for i in range(9)]
ET = sum(REL_EP)

DEN_OFF = [0]
for _nd in REL_ND[:-1]:
    DEN_OFF.append(DEN_OFF[-1] + _nd)
NDEN = DEN_OFF[-1] + REL_ND[-1]
NDENP = ((NDEN + NS * 128 - 1) // (NS * 128)) * (NS * 128)
ZLEN = NDENP // NS


def _pad128(n):
    return ((n + 127) // 128) * 128


NPP, NHP, NZP, NUP = _pad128(NP_), _pad128(NH), _pad128(NZ), _pad128(NU)

ES_COL = {0: 0, 3: 1, 4: 2, 5: 3, 6: 4, 7: 5, 8: 6, 1: 0, 2: 0}
ED_COL = {4: 7, 5: 8, 6: 9, 7: 10, 8: 11, 0: 1, 1: 1, 2: 0, 3: 1}

REL_TAB = {0: (0, 0), 3: (0, 64), 4: (1, 0), 5: (1, 64), 6: (2, 0),
           7: (2, 64), 8: (3, 0), 1: (4, 0), 2: (5, 0)}

_f32 = jnp.float32
_i32 = jnp.int32


def _person_mm_body(x_ref, w_ref, scal_ref, t03_ref, t45_ref, t67_ref,
                    t8_ref):
    res = jnp.dot(x_ref[...], w_ref[...], preferred_element_type=_f32)
    bm = res.shape[0]
    scal_ref[...] = jnp.concatenate(
        [res[:, 448:460], jnp.zeros((bm, 4), _f32)], axis=1)
    t03_ref[...] = res[:, 0:128]
    t45_ref[...] = res[:, 128:256]
    t67_ref[...] = res[:, 256:384]
    t8_ref[...] = jnp.concatenate(
        [res[:, 384:448], jnp.zeros((bm, 64), _f32)], axis=1)


def _person_matmuls(x_person, w_p, bm=1024):
    grid = (pl.cdiv(NP_, bm),)
    outs = ([jax.ShapeDtypeStruct((NP_, 16), _f32)]
            + [jax.ShapeDtypeStruct((NP_, 128), _f32)] * 4)
    specs = ([pl.BlockSpec((bm, 16), lambda i: (i, 0))]
             + [pl.BlockSpec((bm, 128), lambda i: (i, 0))] * 4)
    return pl.pallas_call(
        _person_mm_body,
        grid=grid,
        in_specs=[pl.BlockSpec((bm, 128), lambda i: (i, 0)),
                  pl.BlockSpec((128, 460), lambda i: (0, 0))],
        out_specs=specs,
        out_shape=outs,
    )(x_person, w_p)


def _small_mm_body(ncols, x_ref, w_ref, hs_ref, scal_ref):
    res = jnp.dot(x_ref[...], w_ref[...], preferred_element_type=_f32)
    bm = res.shape[0]
    hs_ref[...] = jnp.concatenate(
        [res[:, 0:64], jnp.zeros((bm, 64), _f32)], axis=1)
    scal_ref[...] = jnp.concatenate(
        [res[:, 64:64 + ncols], jnp.zeros((bm, 16 - ncols), _f32)], axis=1)


def _small_matmul(x, w, ncols, bm=1024):
    m = x.shape[0]
    n = w.shape[1]
    grid = (pl.cdiv(m, bm),)
    return pl.pallas_call(
        functools.partial(_small_mm_body, ncols),
        grid=grid,
        in_specs=[pl.BlockSpec((bm, 128), lambda i: (i, 0)),
                  pl.BlockSpec((128, n), lambda i: (0, 0))],
        out_specs=[pl.BlockSpec((bm, 128), lambda i: (i, 0)),
                   pl.BlockSpec((bm, 16), lambda i: (i, 0))],
        out_shape=[jax.ShapeDtypeStruct((m, 128), _f32),
                   jax.ShapeDtypeStruct((m, 16), _f32)],
    )(x, w)


def _purpose_mm_body(x_ref, w_ref, scal_ref):
    res = jnp.dot(x_ref[...], w_ref[...], preferred_element_type=_f32)
    bm = res.shape[0]
    scal_ref[...] = jnp.concatenate(
        [res[:, 0:2], jnp.zeros((bm, 14), _f32)], axis=1)


def _purpose_matmul(x, w, bm=1024):
    m = x.shape[0]
    return pl.pallas_call(
        _purpose_mm_body,
        grid=(pl.cdiv(m, bm),),
        in_specs=[pl.BlockSpec((bm, 128), lambda i: (i, 0)),
                  pl.BlockSpec((128, 2), lambda i: (0, 0))],
        out_specs=pl.BlockSpec((bm, 16), lambda i: (i, 0)),
        out_shape=jax.ShapeDtypeStruct((m, 16), _f32),
    )(x, w)


_MESH = plsc.VectorSubcoreMesh(core_axis_name="c", subcore_axis_name="s")


def _worker_id():
    c = lax.axis_index("c")
    s = lax.axis_index("s")
    return c, s, s * NC + c


def _chunk_loop(ep, wid, body):
    nch = ep // B
    trip = jnp.maximum((nch - wid + NW - 1) // NW, 0)

    def outer(k, _):
        body((wid + k * NW) * B)
        return 0

    lax.fori_loop(0, trip, outer, 0)


def _zero_vmem_1d(ref, n):
    def zb(j, _):
        ref[pl.ds(j * 16, 16)] = jnp.zeros((16,), _f32)
        return 0
    lax.fori_loop(0, n // 16, zb, 0)


def _copy_1d(src_at, dst_at, total, step):
    k = 0
    while k * step < total:
        ln = min(step, total - k * step)
        pltpu.sync_copy(src_at(k * step, ln), dst_at(k * step, ln))
        k += 1


def _copy_rows_loop(src_at, dst_at, rpt, zr):
    full = rpt // zr
    rem = rpt - full * zr
    if full:
        def cp(k, _):
            pltpu.sync_copy(src_at(k * zr, zr), dst_at(k * zr, zr))
            return 0
        lax.fori_loop(0, full, cp, 0)
    if rem:
        pltpu.sync_copy(src_at(full * zr, rem), dst_at(full * zr, rem))


def _kernel_s_body(*refs):
    (srcs, dsts) = refs[0:9], refs[9:18]
    ess, eds = refs[18:27], refs[27:36]
    ex_out, den0_out, den1_out = refs[36:39]
    (srcbuf, dstbuf, dstobuf, exbuf, esbuf, edbuf, zbuf, den_sh) = refs[39:47]

    c, s, wid = _worker_id()
    iota16 = lax.iota(_i32, 16)

    _zero_vmem_1d(zbuf, 8192)
    _copy_1d(lambda o, ln: zbuf.at[pl.ds(0, ln)],
             lambda o, ln: den_sh.at[pl.ds(s * ZLEN + o, ln)],
             ZLEN, 8192)
    plsc.subcore_barrier()

    for r in range(9):
        ne, off = REL_NE[r], DEN_OFF[r]

        def chunk(lo, r=r, ne=ne, off=off):
            pltpu.sync_copy(srcs[r].at[pl.ds(lo, B)], srcbuf)
            pltpu.sync_copy(dsts[r].at[pl.ds(lo, B)], dstbuf)
            pltpu.sync_copy(ess[r].at[srcbuf], esbuf)
            pltpu.sync_copy(eds[r].at[dstbuf], edbuf)

            def inner(j, _):
                sl = pl.ds(j * 16, 16)
                e = esbuf[sl] + edbuf[sl]
                e = jnp.where(e > 0, e, 0.2 * e)
                ex = jnp.exp(e)
                ex = jnp.where(lo + iota16 + j * 16 < ne, ex, 0.0)
                exbuf[sl] = ex
                dstobuf[sl] = dstbuf[sl] + off
                return 0

            lax.fori_loop(0, B // 16, inner, 0)
            pltpu.sync_copy(exbuf, ex_out.at[pl.ds(REL_EOFF[r] + lo, B)])
            pltpu.sync_copy(exbuf, den_sh.at[dstobuf], add=True)

        _chunk_loop(REL_EP[r], wid, chunk)

    plsc.subcore_barrier()

    @pl.when(c == 0)
    def _():
        _copy_1d(lambda o, ln: den_sh.at[pl.ds(s * ZLEN + o, ln)],
                 lambda o, ln: den0_out.at[pl.ds(s * ZLEN + o, ln)],
                 ZLEN, 8192)

    @pl.when(c == 1)
    def _():
        _copy_1d(lambda o, ln: den_sh.at[pl.ds(s * ZLEN + o, ln)],
                 lambda o, ln: den1_out.at[pl.ds(s * ZLEN + o, ln)],
                 ZLEN, 8192)


def _run_kernel_s(srcs, dsts, ess, eds):
    f = pl.kernel(
        _kernel_s_body,
        out_type=[jax.ShapeDtypeStruct((ET,), _f32),
                  jax.ShapeDtypeStruct((NDENP,), _f32),
                  jax.ShapeDtypeStruct((NDENP,), _f32)],
        mesh=_MESH,
        scratch_types=[
            pltpu.VMEM((B,), _i32), pltpu.VMEM((B,), _i32),
            pltpu.VMEM((B,), _i32), pltpu.VMEM((B,), _f32),
            pltpu.VMEM((B,), _f32), pltpu.VMEM((B,), _f32),
            pltpu.VMEM((8192,), _f32),
            pltpu.VMEM_SHARED((NDENP,), _f32),
        ],
    )
    return f(*srcs, *dsts, *ess, *eds)


def _kernel_da_body(*refs):
    srcs, dsts = refs[0:9], refs[9:18]
    ex_in, den0, den1 = refs[18:21]
    tabs = refs[21:27]
    gouts = refs[27:31]
    (srcbuf, dstbuf, dstobuf, exb, d0b, d1b, ab, rows,
     g0, g1, g2, g3) = refs[31:43]
    gbufs = (g0, g1, g2, g3)

    c, s, wid = _worker_id()

    for r in range(9):
        off = DEN_OFF[r]
        tab_i, half = REL_TAB[r]
        tab = tabs[tab_i]

        def chunk(lo, r=r, off=off, tab=tab, half=half):
            pltpu.sync_copy(srcs[r].at[pl.ds(lo, B)], srcbuf)
            pltpu.sync_copy(dsts[r].at[pl.ds(lo, B)], dstbuf)
            pltpu.sync_copy(ex_in.at[pl.ds(REL_EOFF[r] + lo, B)], exb)

            def addoff(j, _):
                dstobuf[pl.ds(j * 16, 16)] = dstbuf[pl.ds(j * 16, 16)] + off
                return 0

            lax.fori_loop(0, B // 16, addoff, 0)
            pltpu.sync_copy(den0.at[dstobuf], d0b)
            pltpu.sync_copy(den1.at[dstobuf], d1b)
            pltpu.sync_copy(tab.at[srcbuf], rows)

            def scale(j, _):
                sl = pl.ds(j * 16, 16)
                av = exb[sl] / (d0b[sl] + d1b[sl] + 1e-16)
                ab[sl] = av
                for i in range(16):
                    erow = j * 16 + i
                    for c4 in range(4):
                        v = rows[erow, pl.ds(half + 16 * c4, 16)] * av[i]
                        gbufs[c4][pl.ds(256 * j + 16 * i, 16)] = v
                return 0

            lax.fori_loop(0, B // 16, scale, 0)
            gb = (REL_EOFF[r] + lo) * 16
            for c4 in range(4):
                pltpu.sync_copy(gbufs[c4],
                                gouts[c4].at[pl.ds(gb, B * 16)])

        _chunk_loop(REL_EP[r], wid, chunk)


def _run_kernel_da(srcs, dsts, ex_all, den0, den1, tabs):
    f = pl.kernel(
        _kernel_da_body,
        out_type=[jax.ShapeDtypeStruct((ET * 16,), _f32)] * 4,
        mesh=_MESH,
        scratch_types=[
            pltpu.VMEM((B,), _i32), pltpu.VMEM((B,), _i32),
            pltpu.VMEM((B,), _i32), pltpu.VMEM((B,), _f32),
            pltpu.VMEM((B,), _f32), pltpu.VMEM((B,), _f32),
            pltpu.VMEM((B,), _f32), pltpu.VMEM((B, 128), _f32),
            pltpu.VMEM((B * 16,), _f32), pltpu.VMEM((B * 16,), _f32),
            pltpu.VMEM((B * 16,), _f32), pltpu.VMEM((B * 16,), _f32),
        ],
    )
    return f(*srcs, *dsts, ex_all, den0, den1, *tabs)


HROWS = NPP // 2
JROW = HROWS
ACC_ROWS = HROWS + 8


def _kernel_a2_body(phases, nrel, *refs):
    dsts = refs[0:nrel]
    gins = refs[nrel:nrel + 4]
    nouts = len({p[0] for p in phases})
    outs = refs[nrel + 4:nrel + 4 + nouts]
    dstbuf, idxbuf, gflat, zbuf, acc = refs[nrel + 4 + nouts:]

    c, s, wid = _worker_id()
    iota16 = lax.iota(_i32, 16)

    _zero_vmem_1d(zbuf, 8192)

    for oi, base, rows_n, out_off, rel_list in phases:
        rpt16 = (rows_n // NS) * 16
        ndp_out = outs[oi].shape[0] // 32
        _copy_1d(lambda o, ln: zbuf.at[pl.ds(0, ln)],
                 lambda o, ln, s=s, rpt16=rpt16:
                     acc.at[pl.ds(s * rpt16 + o, ln)],
                 rpt16, 8192)
        plsc.subcore_barrier()

        for ri, r, c4 in rel_list:
            def chunk(lo, ri=ri, r=r, c4=c4, base=base, rows_n=rows_n):
                pltpu.sync_copy(dsts[ri].at[pl.ds(lo, B)], dstbuf)
                gb = (REL_EOFF[r] + lo) * 16
                pltpu.sync_copy(gins[c4].at[pl.ds(gb, B * 16)], gflat)

                def mkidx(j, _):
                    sl = pl.ds(j * 16, 16)
                    v = dstbuf[sl]
                    if base is not None:
                        vb = v - base
                        ok = (vb >= 0) & (vb < rows_n)
                        v = jnp.where(ok, vb, JROW)
                    v16 = v * 16
                    for i in range(16):
                        idxbuf[pl.ds(256 * j + 16 * i, 16)] = v16[i] + iota16
                    return 0

                lax.fori_loop(0, B // 16, mkidx, 0)
                pltpu.sync_copy(gflat, acc.at[idxbuf], add=True)

            _chunk_loop(REL_EP[r], wid, chunk)

        plsc.subcore_barrier()
        _copy_1d(lambda o, ln, s=s, rpt16=rpt16:
                     acc.at[pl.ds(s * rpt16 + o, ln)],
                 lambda o, ln, oi=oi, ndp=ndp_out, oo=out_off, s=s,
                        rpt16=rpt16:
                     outs[oi].at[pl.ds((c * ndp + oo) * 16 + s * rpt16 + o,
                                       ln)],
                 rpt16, 8192)
        plsc.subcore_barrier()


def _run_kernel_a2_group(dsts_sub, gins, phases, ndp_out, nouts):
    f = pl.kernel(
        functools.partial(_kernel_a2_body, phases, len(dsts_sub)),
        out_type=[jax.ShapeDtypeStruct((2 * ndp_out * 16,), _f32)] * nouts,
        mesh=_MESH,
        scratch_types=[
            pltpu.VMEM((B,), _i32),
            pltpu.VMEM((B * 16,), _i32),
            pltpu.VMEM((B * 16,), _f32),
            pltpu.VMEM((8192,), _f32),
            pltpu.VMEM_SHARED((ACC_ROWS * 16,), _f32),
        ],
    )
    return f(*dsts_sub, *gins)


def _run_kernel_a2(dsts, gins):
    pe_phases = [(c, hf * HROWS, HROWS, hf * HROWS,
                  [(i, r, c) for i, r in enumerate(range(4, 9))])
                 for c in range(4) for hf in range(2)]
    pe = _run_kernel_a2_group([dsts[r] for r in range(4, 9)], gins,
                              pe_phases, NPP, 4)
    hh = _run_kernel_a2_group([dsts[0]], gins,
                              [(c, None, NHP, 0, [(0, 0, c)])
                               for c in range(4)], NHP, 4)
    zz = _run_kernel_a2_group([dsts[1]], gins,
                              [(c, None, NZP, 0, [(0, 1, c)])
                               for c in range(4)], NZP, 4)
    pu = _run_kernel_a2_group([dsts[2], dsts[3]], gins,
                              [(c, None, NUP, 0, [(0, 2, c), (1, 3, c)])
                               for c in range(4)], NUP, 4)
    return list(pe) + list(hh) + list(zz) + list(pu)


def _epi_body(bias_ref, *refs):
    chunks, o_ref = refs[:-1], refs[-1]
    for ci in range(4):
        n = chunks[ci]
        part = n[0] + n[1]
        o_ref[:, pl.ds(ci * 16, 16)] = jnp.maximum(
            part + bias_ref[0, pl.ds(ci * 16, 16)], 0.0)


def _epilogue(parts, nd, ndp, bias_row, bm=2048):
    parts = [p.reshape(2, ndp, 16) for p in parts]
    return pl.pallas_call(
        _epi_body,
        grid=(pl.cdiv(nd, bm),),
        in_specs=[pl.BlockSpec((1, 64), lambda i: (0, 0))]
        + [pl.BlockSpec((2, bm, 16), lambda i: (0, i, 0))] * 4,
        out_specs=pl.BlockSpec((bm, 64), lambda i: (i, 0)),
        out_shape=jax.ShapeDtypeStruct((nd, 64), _f32),
    )(bias_row, *parts)


def kernel(x_person, x_household, x_zone, x_purpose, ei_belongs_to,
           ei_located_in, ei_has_purpose, ei_performs, ei_parent, ei_child,
           ei_spouse, ei_housemate, ei_sibling, Ws, Wd, att_s, att_d, bias):
    eis = [ei_belongs_to, ei_located_in, ei_has_purpose, ei_performs,
           ei_parent, ei_child, ei_spouse, ei_housemate, ei_sibling]

    vs = jnp.einsum('rdh,rh->rd', Ws, att_s)
    vd = jnp.einsum('rdh,rh->rd', Wd, att_d)

    w_p = jnp.concatenate(
        [Ws[i] for i in PERSON_SRC_RELS]
        + [vs[jnp.array(PERSON_SRC_RELS)].T, vd[4:9].T], axis=1)
    w_h = jnp.concatenate([Ws[1], vs[1][:, None], vd[0][:, None]], axis=1)
    w_z = jnp.concatenate([Ws[2], vs[2][:, None], vd[1][:, None]], axis=1)
    w_u = vd[2:4].T

    t_p, t03, t45, t67, t8 = _person_matmuls(x_person, w_p)
    th, t_h = _small_matmul(x_household, w_h, 2)
    tz, t_z = _small_matmul(x_zone, w_z, 2)
    t_u = _purpose_matmul(x_purpose, w_u)
    tabs = [t03, t45, t67, t8, th, tz]

    srcs, dsts = [], []
    for r in range(9):
        pad = REL_EP[r] - REL_NE[r]
        srcs.append(jnp.concatenate([eis[r][0], jnp.zeros((pad,), _i32)]))
        dsts.append(jnp.concatenate([eis[r][1], jnp.zeros((pad,), _i32)]))

    scal_src = [t_p, t_h, t_z, t_p, t_p, t_p, t_p, t_p, t_p]
    scal_dst = [t_h, t_z, t_u, t_u, t_p, t_p, t_p, t_p, t_p]
    ess = [scal_src[r][:, ES_COL[r]] for r in range(9)]
    eds = [scal_dst[r][:, ED_COL[r]] for r in range(9)]

    ex_all, den0, den1 = _run_kernel_s(srcs, dsts, ess, eds)
    gchunks = _run_kernel_da(srcs, dsts, ex_all, den0, den1, tabs)
    nums = _run_kernel_a2(dsts, gchunks)

    bias_pe = jnp.sum(bias[4:9], axis=0)[None, :]
    pe = _epilogue(list(nums[0:4]), NP_, NPP, bias_pe)
    hh = _epilogue(list(nums[4:8]), NH, NHP, bias[0][None, :])
    zz = _epilogue(list(nums[8:12]), NZ, NZP, bias[1][None, :])
    pu = _epilogue(list(nums[12:16]), NU, NUP, (bias[2] + bias[3])[None, :])
    return (pe, hh, zz, pu)
